# baseline (speedup 1.0000x reference)
import functools

import jax
import jax.numpy as jnp
from jax import lax
from jax.experimental import pallas as pl
from jax.experimental.pallas import tpu as pltpu
from jax.experimental.pallas import tpu_sc as plsc

_N = 4096
_K = 8192
_D = 256
_T = 256
_GRID = _N // _T


def _vq_body(x_ref, x2_ref, e2x_ref, e2_ref, cols_ref, idx_ref, loss_ref):
    g = pl.program_id(0)
    xb = x_ref[...]
    x2 = x2_ref[...].reshape(1, _T)
    e2 = e2_ref[...]
    m = lax.dot_general(
        e2x_ref[...], xb, (((1,), (0,)), ((), ())),
        preferred_element_type=jnp.float32)
    c = (x2 + e2) - (m + m)
    d2 = jnp.maximum(c, jnp.float32(0.0))
    d = jnp.sqrt(d2)
    dmin = jnp.min(d, axis=0, keepdims=True)
    colsf = cols_ref[...]
    big = jnp.float32(1e9)
    idxf = jnp.min(jnp.where(d == dmin, colsf, big), axis=0)
    idx_ref[...] = idxf.reshape(1, 1, _T).astype(jnp.int32)
    part = jnp.sum(dmin * dmin).reshape(1, 1)

    @pl.when(g == 0)
    def _init():
        loss_ref[...] = jnp.zeros((1, 1), jnp.float32)

    loss_ref[...] += part


def _vq_tc(x_r, x2_r, e2x, e2, colsf):
    nt = 1024 // _T
    return pl.pallas_call(
        _vq_body,
        grid=(_GRID,),
        in_specs=[
            pl.BlockSpec((_D, _T), lambda g: (g // nt, g % nt)),
            pl.BlockSpec((1, 1, _T), lambda g: (g, 0, 0)),
            pl.BlockSpec((_K, _D), lambda g: (0, 0)),
            pl.BlockSpec((_K, 1), lambda g: (0, 0)),
            pl.BlockSpec((_K, 1), lambda g: (0, 0)),
        ],
        out_specs=[
            pl.BlockSpec((1, 1, _T), lambda g: (g, 0, 0)),
            pl.BlockSpec((1, 1), lambda g: (0, 0)),
        ],
        out_shape=[
            jax.ShapeDtypeStruct((_GRID, 1, _T), jnp.int32),
            jax.ShapeDtypeStruct((1, 1), jnp.float32),
        ],
        compiler_params=pltpu.CompilerParams(
            dimension_semantics=("arbitrary",),
        ),
    )(x_r, x2_r, e2x, e2, colsf)


def _make_sc_gather():
    info = plsc.get_sparse_core_info()
    nc, ns = info.num_cores, info.num_subcores
    nw = nc * ns
    b_per_w = _N // nw
    mesh = plsc.VectorSubcoreMesh(core_axis_name="c", subcore_axis_name="s")

    @functools.partial(
        pl.kernel,
        mesh=mesh,
        out_type=jax.ShapeDtypeStruct((_N, _D), jnp.float32),
        scratch_types=[
            pltpu.VMEM((b_per_w,), jnp.int32),
            pltpu.VMEM((b_per_w, _D), jnp.float32),
            pltpu.SemaphoreType.DMA,
        ],
    )
    def gather_k(table_hbm, idx_hbm, out_hbm, idx_v, rows_v, sem):
        wid = lax.axis_index("s") * nc + lax.axis_index("c")
        base = wid * b_per_w
        pltpu.sync_copy(idx_hbm.at[pl.ds(base, b_per_w)], idx_v)
        pltpu.async_copy(table_hbm.at[idx_v], rows_v, sem).wait()
        pltpu.sync_copy(rows_v, out_hbm.at[pl.ds(base, b_per_w)])

    return gather_k


def kernel(inputs, E):
    B, C, L, H, W = inputs.shape
    K, D = E.shape
    x = jnp.transpose(inputs, (0, 2, 3, 4, 1)).reshape(-1, D)
    xf = x.astype(jnp.float32)
    ef = E.astype(jnp.float32)
    x2 = jnp.sum(xf * xf, axis=1, keepdims=True)
    e2 = jnp.sum(ef * ef, axis=1)[:, None]

    x_r = inputs.reshape(B * C, L * H * W)
    x2_r = x2.reshape(_GRID, 1, _T)
    colsf = lax.iota(jnp.float32, _K)[:, None]
    idx3d, loss_raw = _vq_tc(x_r, x2_r, ef, e2, colsf)
    idx = idx3d.reshape(-1)

    quant_rows = _make_sc_gather()(ef, idx)

    quant = quant_rows.reshape(B, L, H, W, C)
    quant_st = jnp.transpose(quant, (0, 4, 1, 2, 3))
    loss = (loss_raw[0, 0] * jnp.float32(1.25)) / jnp.float32(_N * _D)
    return quant_st, loss, idx[:, None]


# speedup vs baseline: 1.1115x; 1.1115x over previous
import functools

import jax
import jax.numpy as jnp
from jax import lax
from jax.experimental import pallas as pl
from jax.experimental.pallas import tpu as pltpu
from jax.experimental.pallas import tpu_sc as plsc

_N = 4096
_K = 8192
_D = 256
_T = 512
_GRID = _N // _T


def _vq_body(x_ref, x2_ref, e2x_ref, e2_ref, cols_ref, idx_ref, loss_ref):
    g = pl.program_id(0)
    xb = x_ref[...]
    x2 = x2_ref[...].reshape(1, _T)
    e2 = e2_ref[...]
    m2 = lax.dot_general(
        e2x_ref[...], xb, (((1,), (0,)), ((), ())),
        preferred_element_type=jnp.float32)
    c = (x2 + e2) - m2
    d2 = jnp.maximum(c, jnp.float32(0.0))
    d = jnp.sqrt(d2)
    dmin = jnp.min(d, axis=0, keepdims=True)
    colsf = cols_ref[...]
    big = jnp.float32(1e9)
    idxf = jnp.min(jnp.where(d == dmin, colsf, big), axis=0)
    idx_ref[...] = idxf.reshape(1, 1, _T).astype(jnp.int32)
    part = jnp.sum(dmin * dmin).reshape(1, 1)

    @pl.when(g == 0)
    def _init():
        loss_ref[...] = jnp.zeros((1, 1), jnp.float32)

    loss_ref[...] += part


def _vq_tc(x_r, x2_r, e2x, e2, colsf):
    nt = 1024 // _T
    return pl.pallas_call(
        _vq_body,
        grid=(_GRID,),
        in_specs=[
            pl.BlockSpec((_D, _T), lambda g: (g // nt, g % nt)),
            pl.BlockSpec((1, 1, _T), lambda g: (g, 0, 0)),
            pl.BlockSpec((_K, _D), lambda g: (0, 0)),
            pl.BlockSpec((_K, 1), lambda g: (0, 0)),
            pl.BlockSpec((_K, 1), lambda g: (0, 0)),
        ],
        out_specs=[
            pl.BlockSpec((1, 1, _T), lambda g: (g, 0, 0)),
            pl.BlockSpec((1, 1), lambda g: (0, 0)),
        ],
        out_shape=[
            jax.ShapeDtypeStruct((_GRID, 1, _T), jnp.int32),
            jax.ShapeDtypeStruct((1, 1), jnp.float32),
        ],
        compiler_params=pltpu.CompilerParams(
            dimension_semantics=("arbitrary",),
        ),
    )(x_r, x2_r, e2x, e2, colsf)


def _make_sc_gather():
    info = plsc.get_sparse_core_info()
    nc, ns = info.num_cores, info.num_subcores
    nw = nc * ns
    b_per_w = _N // nw
    mesh = plsc.VectorSubcoreMesh(core_axis_name="c", subcore_axis_name="s")

    @functools.partial(
        pl.kernel,
        mesh=mesh,
        out_type=jax.ShapeDtypeStruct((_N, _D), jnp.float32),
        scratch_types=[
            pltpu.VMEM((b_per_w,), jnp.int32),
            pltpu.VMEM((b_per_w, _D), jnp.float32),
            pltpu.SemaphoreType.DMA,
        ],
    )
    def gather_k(table_hbm, idx_hbm, out_hbm, idx_v, rows_v, sem):
        wid = lax.axis_index("s") * nc + lax.axis_index("c")
        base = wid * b_per_w
        pltpu.sync_copy(idx_hbm.at[pl.ds(base, b_per_w)], idx_v)
        pltpu.async_copy(table_hbm.at[idx_v], rows_v, sem).wait()
        pltpu.sync_copy(rows_v, out_hbm.at[pl.ds(base, b_per_w)])

    return gather_k


def kernel(inputs, E):
    B, C, L, H, W = inputs.shape
    K, D = E.shape
    x = jnp.transpose(inputs, (0, 2, 3, 4, 1)).reshape(-1, D)
    xf = x.astype(jnp.float32)
    ef = E.astype(jnp.float32)
    x2 = jnp.sum(xf * xf, axis=1, keepdims=True)
    e2 = jnp.sum(ef * ef, axis=1)[:, None]

    x_r = inputs.reshape(B * C, L * H * W)
    x2_r = x2.reshape(_GRID, 1, _T)
    colsf = lax.iota(jnp.float32, _K)[:, None]
    idx3d, loss_raw = _vq_tc(x_r, x2_r, ef + ef, e2, colsf)
    idx = idx3d.reshape(-1)

    quant_rows = _make_sc_gather()(ef, idx)

    quant = quant_rows.reshape(B, L, H, W, C)
    quant_st = jnp.transpose(quant, (0, 4, 1, 2, 3))
    loss = (loss_raw[0, 0] * jnp.float32(1.25)) / jnp.float32(_N * _D)
    return quant_st, loss, idx[:, None]


# speedup vs baseline: 1.1255x; 1.0126x over previous
import functools

import jax
import jax.numpy as jnp
from jax import lax
from jax.experimental import pallas as pl
from jax.experimental.pallas import tpu as pltpu
from jax.experimental.pallas import tpu_sc as plsc

_N = 4096
_K = 8192
_D = 256
_T = 1024
_GRID = _N // _T


def _vq_body(x_ref, x2_ref, e2x_ref, e2_ref, cols_ref, idx_ref, loss_ref):
    g = pl.program_id(0)
    xb = x_ref[...]
    x2 = x2_ref[...].reshape(1, _T)
    e2 = e2_ref[...]
    m2 = lax.dot_general(
        e2x_ref[...], xb, (((1,), (0,)), ((), ())),
        preferred_element_type=jnp.float32)
    c = (x2 + e2) - m2
    d2 = jnp.maximum(c, jnp.float32(0.0))
    d = jnp.sqrt(d2)
    dmin = jnp.min(d, axis=0, keepdims=True)
    colsf = cols_ref[...]
    big = jnp.float32(1e9)
    idxf = jnp.min(jnp.where(d == dmin, colsf, big), axis=0)
    idx_ref[...] = idxf.reshape(1, 1, _T).astype(jnp.int32)
    part = jnp.sum(dmin * dmin).reshape(1, 1)

    @pl.when(g == 0)
    def _init():
        loss_ref[...] = jnp.zeros((1, 1), jnp.float32)

    loss_ref[...] += part


def _vq_tc(x_r, x2_r, e2x, e2, colsf):
    nt = 1024 // _T
    return pl.pallas_call(
        _vq_body,
        grid=(_GRID,),
        in_specs=[
            pl.BlockSpec((_D, _T), lambda g: (g // nt, g % nt)),
            pl.BlockSpec((1, 1, _T), lambda g: (g, 0, 0)),
            pl.BlockSpec((_K, _D), lambda g: (0, 0)),
            pl.BlockSpec((_K, 1), lambda g: (0, 0)),
            pl.BlockSpec((_K, 1), lambda g: (0, 0)),
        ],
        out_specs=[
            pl.BlockSpec((1, 1, _T), lambda g: (g, 0, 0)),
            pl.BlockSpec((1, 1), lambda g: (0, 0)),
        ],
        out_shape=[
            jax.ShapeDtypeStruct((_GRID, 1, _T), jnp.int32),
            jax.ShapeDtypeStruct((1, 1), jnp.float32),
        ],
        compiler_params=pltpu.CompilerParams(
            dimension_semantics=("arbitrary",),
        ),
    )(x_r, x2_r, e2x, e2, colsf)


def _make_sc_gather():
    info = plsc.get_sparse_core_info()
    nc, ns = info.num_cores, info.num_subcores
    nw = nc * ns
    b_per_w = _N // nw
    mesh = plsc.VectorSubcoreMesh(core_axis_name="c", subcore_axis_name="s")

    @functools.partial(
        pl.kernel,
        mesh=mesh,
        out_type=jax.ShapeDtypeStruct((_N, _D), jnp.float32),
        scratch_types=[
            pltpu.VMEM((b_per_w,), jnp.int32),
            pltpu.VMEM((b_per_w, _D), jnp.float32),
            pltpu.SemaphoreType.DMA,
        ],
    )
    def gather_k(table_hbm, idx_hbm, out_hbm, idx_v, rows_v, sem):
        wid = lax.axis_index("s") * nc + lax.axis_index("c")
        base = wid * b_per_w
        pltpu.sync_copy(idx_hbm.at[pl.ds(base, b_per_w)], idx_v)
        pltpu.async_copy(table_hbm.at[idx_v], rows_v, sem).wait()
        pltpu.sync_copy(rows_v, out_hbm.at[pl.ds(base, b_per_w)])

    return gather_k


def kernel(inputs, E):
    B, C, L, H, W = inputs.shape
    K, D = E.shape
    x = jnp.transpose(inputs, (0, 2, 3, 4, 1)).reshape(-1, D)
    xf = x.astype(jnp.float32)
    ef = E.astype(jnp.float32)
    x2 = jnp.sum(xf * xf, axis=1, keepdims=True)
    e2 = jnp.sum(ef * ef, axis=1)[:, None]

    x_r = inputs.reshape(B * C, L * H * W)
    x2_r = x2.reshape(_GRID, 1, _T)
    colsf = lax.iota(jnp.float32, _K)[:, None]
    idx3d, loss_raw = _vq_tc(x_r, x2_r, ef + ef, e2, colsf)
    idx = idx3d.reshape(-1)

    quant_rows = _make_sc_gather()(ef, idx)

    quant = quant_rows.reshape(B, L, H, W, C)
    quant_st = jnp.transpose(quant, (0, 4, 1, 2, 3))
    loss = (loss_raw[0, 0] * jnp.float32(1.25)) / jnp.float32(_N * _D)
    return quant_st, loss, idx[:, None]
